# Initial kernel scaffold; baseline (speedup 1.0000x reference)
import jax
import jax.numpy as jnp
from jax.experimental import pallas as pl


def kernel(node_features, edge_features, edge_flats, graph, W1, root1, b1, W2, root2, b2):
    raise NotImplementedError("write your pallas kernel here")


# speedup vs baseline: 9.2438x; 9.2438x over previous
import functools

import jax
import jax.numpy as jnp
from jax import lax
from jax.experimental import pallas as pl
from jax.experimental.pallas import tpu as pltpu
from jax.experimental.pallas import tpu_sc as plsc

N = 10000
E = 320000
R = 4
NS = 16
NC = 2
CH = 80
EPS = 20480
RPS = EPS // CH
GR = 8
IGR = 32
NGR = RPS // GR
RPW = RPS * NS // (NS * NC)
NPS = 632
NPAD = NPS * NS
N4P = 40960
TS = 1000

_mesh = plsc.VectorSubcoreMesh(
    core_axis_name="c", subcore_axis_name="s", num_cores=NC, num_subcores=NS)


def _prep_body(ef_ref, src_ref, dst_ref, gidx_ref, didx_ref):
    m = ef_ref[0]
    lab = jnp.zeros(m.shape, jnp.int32)
    for j in range(1, R):
        v = ef_ref[j]
        take = v > m
        lab = jnp.where(take, j, lab)
        m = jnp.where(take, v, m)
    gidx_ref[...] = lab * N + src_ref[...]
    didx_ref[...] = dst_ref[...] * R + lab


def _prep(efT3, src2d, dst2d):
    return pl.pallas_call(
        _prep_body,
        out_shape=[
            jax.ShapeDtypeStruct(src2d.shape, jnp.int32),
            jax.ShapeDtypeStruct(src2d.shape, jnp.int32),
        ],
    )(efT3, src2d, dst2d)


def _mm_body(h_ref, w_ref, b_ref, out_ref):
    r = pl.program_id(0)
    kc = pl.program_id(3)
    x = h_ref[0]
    part = jnp.dot(x, w_ref[0, 0, 0], preferred_element_type=jnp.float32)

    @pl.when(kc == 0)
    def _():
        scale = (r == R).astype(jnp.float32)
        out_ref[0] = part + scale * b_ref[0, 0][None, :]

    @pl.when(kc != 0)
    def _():
        out_ref[0] += part


def _mm(hin, w5, b3d):
    kcn = hin.shape[0]
    return pl.pallas_call(
        _mm_body,
        grid=(R + 1, N // TS, 2, kcn),
        in_specs=[
            pl.BlockSpec((1, TS, 128), lambda r, si, c, kc: (kc, si, 0)),
            pl.BlockSpec((1, 1, 1, 128, 128), lambda r, si, c, kc: (c, r, kc, 0, 0)),
            pl.BlockSpec((1, 1, 128), lambda r, si, c, kc: (c, 0, 0)),
        ],
        out_specs=pl.BlockSpec(
            (1, TS, 128), lambda r, si, c, kc: (c, r * (N // TS) + si, 0)),
        out_shape=jax.ShapeDtypeStruct((2, (R + 1) * N, 128), jnp.float32),
        compiler_params=pltpu.CompilerParams(
            dimension_semantics=("parallel", "parallel", "parallel", "arbitrary")),
    )(hin, w5, b3d)


def _cnt_body(didx16_hbm, didx32_hbm, norm_hbm,
              cnt_sh, didx_v, norm_v, vals_v, zeros_v, ones_v):
    c = lax.axis_index("c")
    s = lax.axis_index("s")

    def fill(i, _):
        zeros_v[pl.ds(i * 16, 16)] = jnp.zeros((16,), jnp.float32)
        return 0
    lax.fori_loop(0, N4P // NS // 16, fill, 0)
    for t in range(CH // 16):
        ones_v[pl.ds(t * 16, 16)] = jnp.ones((16,), jnp.float32)

    off = pl.multiple_of(s * (N4P // NS), 8)
    pltpu.sync_copy(zeros_v, cnt_sh.at[pl.ds(off, N4P // NS)])
    plsc.subcore_barrier()

    def cgroup(gg, _):
        pltpu.sync_copy(didx16_hbm.at[s, pl.ds(gg * GR, GR)], didx_v)

        def chunk(kk, _2):
            pltpu.sync_copy(ones_v, cnt_sh.at[didx_v.at[kk]], add=True)
            return 0
        lax.fori_loop(0, GR, chunk, 0)
        return 0
    lax.fori_loop(0, NGR, cgroup, 0)
    plsc.subcore_barrier()

    g = s * NC + c

    def ngroup(gg, _):
        pltpu.sync_copy(didx32_hbm.at[g, pl.ds(gg * GR, GR)], didx_v)

        def nrow(kk, _2):
            pltpu.sync_copy(cnt_sh.at[didx_v.at[kk]], vals_v)
            for t in range(CH // 16):
                v = vals_v[pl.ds(t * 16, 16)]
                real = didx_v[kk, pl.ds(t * 16, 16)] < N * R
                norm_v[kk, pl.ds(t * 16, 16)] = jnp.where(
                    real, 1.0 / jnp.maximum(v, 1.0), 0.0)
            return 0
        lax.fori_loop(0, GR, nrow, 0)
        pltpu.sync_copy(norm_v, norm_hbm.at[g, pl.ds(gg * GR, GR)])
        return 0
    lax.fori_loop(0, RPW // GR, ngroup, 0)


_cnt_call = pl.kernel(
    _cnt_body,
    out_type=jax.ShapeDtypeStruct((NS * NC, RPW, CH), jnp.float32),
    mesh=_mesh,
    scratch_types=[
        pltpu.VMEM_SHARED((N4P,), jnp.float32),
        pltpu.VMEM((GR, CH), jnp.int32),
        pltpu.VMEM((GR, CH), jnp.float32),
        pltpu.VMEM((CH,), jnp.float32),
        pltpu.VMEM((N4P // NS,), jnp.float32),
        pltpu.VMEM((CH,), jnp.float32),
    ],
)


def _bcast16(vec16, e):
    dnums = lax.GatherDimensionNumbers(
        offset_dims=(), collapsed_slice_dims=(0,), start_index_map=(0,))
    idx = jnp.full((16, 1), e, jnp.int32)
    return lax.gather(vec16, idx, dnums, slice_sizes=(1,),
                      mode=lax.GatherScatterMode.PROMISE_IN_BOUNDS)


def _agg_body(xt_hbm, gidx_hbm, dst_hbm, norm_hbm, h_hbm,
              acc, gidx_v, dst_v, nrm_v, rows_v, drain_v):
    c = lax.axis_index("c")
    s = lax.axis_index("s")
    base = pl.multiple_of(s * NPS, 8)

    @pl.when(s < NS - 1)
    def _():
        pltpu.sync_copy(xt_hbm.at[c, pl.ds(pl.multiple_of(R * N + base, 8), NPS), :],
                        acc.at[pl.ds(base, NPS)])

    @pl.when(s == NS - 1)
    def _():
        tail = N - (NS - 1) * NPS
        pltpu.sync_copy(xt_hbm.at[c, pl.ds(pl.multiple_of(R * N + base, 8), tail), :],
                        acc.at[pl.ds(base, tail)])

    plsc.subcore_barrier()

    def ggroup(g, _0):
        gb = pl.multiple_of(g * IGR, 8)
        pltpu.sync_copy(gidx_hbm.at[s, pl.ds(gb, IGR)], gidx_v)
        pltpu.sync_copy(dst_hbm.at[s, pl.ds(gb, IGR)], dst_v)
        pltpu.sync_copy(norm_hbm.at[s, pl.ds(gb, IGR)], nrm_v)

        def chunk(k, _):
            pltpu.sync_copy(xt_hbm.at[c].at[gidx_v.at[k]], rows_v)

            def grp(gi, _2):
                nrm16 = nrm_v[k, pl.ds(gi * 16, 16)]
                for e in range(16):
                    nb = _bcast16(nrm16, e)
                    j = gi * 16 + e
                    for t in range(8):
                        rows_v[j, pl.ds(t * 16, 16)] = (
                            rows_v[j, pl.ds(t * 16, 16)] * nb)
                return 0
            lax.fori_loop(0, CH // 16, grp, 0)
            pltpu.sync_copy(rows_v, acc.at[dst_v.at[k]], add=True)
            return 0
        lax.fori_loop(0, IGR, chunk, 0)
        return 0
    lax.fori_loop(0, RPS // IGR, ggroup, 0)
    plsc.subcore_barrier()

    def drain_chunk(start, size):
        rb = pl.multiple_of(base + start, 8)
        pltpu.sync_copy(acc.at[pl.ds(rb, size)], drain_v.at[pl.ds(0, size)])

        def rrow(r2, _3):
            for u in range(8):
                drain_v[r2, pl.ds(u * 16, 16)] = jnp.maximum(
                    drain_v[r2, pl.ds(u * 16, 16)], 0.0)
            return 0
        lax.fori_loop(0, size, rrow, 0)
        pltpu.sync_copy(drain_v.at[pl.ds(0, size)], h_hbm.at[c, pl.ds(rb, size), :])

    for t in range(3):
        drain_chunk(t * 160, 160)

    @pl.when(s < NS - 1)
    def _():
        drain_chunk(480, NPS - 480)

    @pl.when(s == NS - 1)
    def _():
        drain_chunk(480, N - (NS - 1) * NPS - 480)


_agg_call = pl.kernel(
    _agg_body,
    out_type=jax.ShapeDtypeStruct((2, N, 128), jnp.float32),
    mesh=_mesh,
    scratch_types=[
        pltpu.VMEM_SHARED((NPAD, 128), jnp.float32),
        pltpu.VMEM((IGR, CH), jnp.int32),
        pltpu.VMEM((IGR, CH), jnp.int32),
        pltpu.VMEM((IGR, CH), jnp.float32),
        pltpu.VMEM((CH, 128), jnp.float32),
        pltpu.VMEM((160, 128), jnp.float32),
    ],
)


def kernel(node_features, edge_features, edge_flats, graph,
           W1, root1, b1, W2, root2, b2):
    del graph
    src = edge_flats[0].astype(jnp.int32)
    dst = edge_flats[1].astype(jnp.int32)

    efT3 = edge_features.T.reshape(8, E // 128, 128)
    gidx2d, didx2d = _prep(efT3, src.reshape(E // 128, 128),
                           dst.reshape(E // 128, 128))
    npad = EPS * NS - E
    gidx = jnp.concatenate(
        [gidx2d.reshape(-1), jnp.zeros((npad,), jnp.int32)]).reshape(NS, RPS, CH)
    didx_flat = jnp.concatenate(
        [didx2d.reshape(-1), jnp.full((npad,), N * R, jnp.int32)])
    didx16 = didx_flat.reshape(NS, RPS, CH)
    didx32 = didx_flat.reshape(NS * NC, RPW, CH)
    dst80 = jnp.concatenate(
        [dst, jnp.zeros((npad,), jnp.int32)]).reshape(NS, RPS, CH)

    norm = _cnt_call(didx16, didx32).reshape(NS, RPS, CH)

    w1e = (jnp.concatenate([W1, root1[None]], axis=0)
           .reshape(R + 1, 1, 128, 2, 128).transpose(3, 0, 1, 2, 4))
    xt1 = _mm(node_features.reshape(1, N, 128), w1e, b1.reshape(2, 1, 128))
    h1 = _agg_call(xt1, gidx, dst80, norm)

    w2e = (jnp.concatenate([W2, root2[None]], axis=0)
           .reshape(R + 1, 2, 128, 2, 128).transpose(3, 0, 1, 2, 4))
    xt2 = _mm(h1, w2e, b2.reshape(2, 1, 128))
    h2 = _agg_call(xt2, gidx, dst80, norm)

    return (h2.transpose(1, 0, 2).reshape(N, 256), edge_features)


# speedup vs baseline: 11.4290x; 1.2364x over previous
import functools

import jax
import jax.numpy as jnp
from jax import lax
from jax.experimental import pallas as pl
from jax.experimental.pallas import tpu as pltpu
from jax.experimental.pallas import tpu_sc as plsc

N = 10000
E = 320000
R = 4
NS = 16
NC = 2
CH = 80
EPS = 20480
RPS = EPS // CH
GR = 8
IGR = 32
NGR = RPS // GR
RPW = RPS * NS // (NS * NC)
NPS = 632
NPAD = NPS * NS
N4P = 40960
TS = 1000

_mesh = plsc.VectorSubcoreMesh(
    core_axis_name="c", subcore_axis_name="s", num_cores=NC, num_subcores=NS)


def _prep_body(ef_ref, src_ref, dst_ref, gidx_ref, didx_ref):
    m = ef_ref[0]
    lab = jnp.zeros(m.shape, jnp.int32)
    for j in range(1, R):
        v = ef_ref[j]
        take = v > m
        lab = jnp.where(take, j, lab)
        m = jnp.where(take, v, m)
    gidx_ref[...] = lab * N + src_ref[...]
    didx_ref[...] = dst_ref[...] * R + lab


def _prep(efT3, src2d, dst2d):
    return pl.pallas_call(
        _prep_body,
        out_shape=[
            jax.ShapeDtypeStruct(src2d.shape, jnp.int32),
            jax.ShapeDtypeStruct(src2d.shape, jnp.int32),
        ],
    )(efT3, src2d, dst2d)


def _mm_body(h_ref, w_ref, b_ref, out_ref):
    r = pl.program_id(0)
    kc = pl.program_id(3)
    x = h_ref[0]
    part = jnp.dot(x, w_ref[0, 0, 0], preferred_element_type=jnp.float32)

    @pl.when(kc == 0)
    def _():
        scale = (r == R).astype(jnp.float32)
        out_ref[0] = part + scale * b_ref[0, 0][None, :]

    @pl.when(kc != 0)
    def _():
        out_ref[0] += part


def _mm(hin, w5, b3d):
    kcn = hin.shape[0]
    return pl.pallas_call(
        _mm_body,
        grid=(R + 1, N // TS, 2, kcn),
        in_specs=[
            pl.BlockSpec((1, TS, 128), lambda r, si, c, kc: (kc, si, 0)),
            pl.BlockSpec((1, 1, 1, 128, 128), lambda r, si, c, kc: (c, r, kc, 0, 0)),
            pl.BlockSpec((1, 1, 128), lambda r, si, c, kc: (c, 0, 0)),
        ],
        out_specs=pl.BlockSpec(
            (1, TS, 128), lambda r, si, c, kc: (c, r * (N // TS) + si, 0)),
        out_shape=jax.ShapeDtypeStruct((2, (R + 1) * N, 128), jnp.float32),
        compiler_params=pltpu.CompilerParams(
            dimension_semantics=("parallel", "parallel", "parallel", "arbitrary")),
    )(hin, w5, b3d)


def _cnt_body(didx16_hbm, didx32_hbm, norm_hbm,
              cnt_sh, didx_v, norm_v, vals_v, zeros_v, ones_v):
    c = lax.axis_index("c")
    s = lax.axis_index("s")

    def fill(i, _):
        zeros_v[pl.ds(i * 16, 16)] = jnp.zeros((16,), jnp.float32)
        return 0
    lax.fori_loop(0, N4P // NS // 16, fill, 0)
    for t in range(CH // 16):
        ones_v[pl.ds(t * 16, 16)] = jnp.ones((16,), jnp.float32)

    off = pl.multiple_of(s * (N4P // NS), 8)
    pltpu.sync_copy(zeros_v, cnt_sh.at[pl.ds(off, N4P // NS)])
    plsc.subcore_barrier()

    def cgroup(gg, _):
        pltpu.sync_copy(didx16_hbm.at[s, pl.ds(gg * GR, GR)], didx_v)

        def chunk(kk, _2):
            pltpu.sync_copy(ones_v, cnt_sh.at[didx_v.at[kk]], add=True)
            return 0
        lax.fori_loop(0, GR, chunk, 0)
        return 0
    lax.fori_loop(0, NGR, cgroup, 0)
    plsc.subcore_barrier()

    g = s * NC + c

    def ngroup(gg, _):
        pltpu.sync_copy(didx32_hbm.at[g, pl.ds(gg * GR, GR)], didx_v)

        def nrow(kk, _2):
            pltpu.sync_copy(cnt_sh.at[didx_v.at[kk]], vals_v)
            for t in range(CH // 16):
                v = vals_v[pl.ds(t * 16, 16)]
                real = didx_v[kk, pl.ds(t * 16, 16)] < N * R
                norm_v[kk, pl.ds(t * 16, 16)] = jnp.where(
                    real, 1.0 / jnp.maximum(v, 1.0), 0.0)
            return 0
        lax.fori_loop(0, GR, nrow, 0)
        pltpu.sync_copy(norm_v, norm_hbm.at[g, pl.ds(gg * GR, GR)])
        return 0
    lax.fori_loop(0, RPW // GR, ngroup, 0)


_cnt_call = pl.kernel(
    _cnt_body,
    out_type=jax.ShapeDtypeStruct((NS * NC, RPW, CH), jnp.float32),
    mesh=_mesh,
    scratch_types=[
        pltpu.VMEM_SHARED((N4P,), jnp.float32),
        pltpu.VMEM((GR, CH), jnp.int32),
        pltpu.VMEM((GR, CH), jnp.float32),
        pltpu.VMEM((CH,), jnp.float32),
        pltpu.VMEM((N4P // NS,), jnp.float32),
        pltpu.VMEM((CH,), jnp.float32),
    ],
)


def _bcast16(vec16, e):
    dnums = lax.GatherDimensionNumbers(
        offset_dims=(), collapsed_slice_dims=(0,), start_index_map=(0,))
    idx = jnp.full((16, 1), e, jnp.int32)
    return lax.gather(vec16, idx, dnums, slice_sizes=(1,),
                      mode=lax.GatherScatterMode.PROMISE_IN_BOUNDS)


def _agg_body(xt_hbm, gidx_hbm, dst_hbm, norm_hbm, h_hbm,
              acc, gidx_v, dst_v, nrm_v, rows0, rows1, drain_v,
              gsem0, gsem1):
    c = lax.axis_index("c")
    s = lax.axis_index("s")
    base = pl.multiple_of(s * NPS, 8)

    @pl.when(s < NS - 1)
    def _():
        pltpu.sync_copy(xt_hbm.at[c, pl.ds(pl.multiple_of(R * N + base, 8), NPS), :],
                        acc.at[pl.ds(base, NPS)])

    @pl.when(s == NS - 1)
    def _():
        tail = N - (NS - 1) * NPS
        pltpu.sync_copy(xt_hbm.at[c, pl.ds(pl.multiple_of(R * N + base, 8), tail), :],
                        acc.at[pl.ds(base, tail)])

    plsc.subcore_barrier()

    def scale_scatter(rows_v, k):
        def grp(gi, _2):
            nrm16 = nrm_v[k, pl.ds(gi * 16, 16)]
            for e in range(16):
                nb = _bcast16(nrm16, e)
                j = gi * 16 + e
                for t in range(8):
                    rows_v[j, pl.ds(t * 16, 16)] = (
                        rows_v[j, pl.ds(t * 16, 16)] * nb)
            return 0
        lax.fori_loop(0, CH // 16, grp, 0)
        pltpu.sync_copy(rows_v, acc.at[dst_v.at[k]], add=True)

    def ggroup(g, _0):
        gb = pl.multiple_of(g * IGR, 8)
        pltpu.sync_copy(gidx_hbm.at[s, pl.ds(gb, IGR)], gidx_v)
        pltpu.sync_copy(dst_hbm.at[s, pl.ds(gb, IGR)], dst_v)
        pltpu.sync_copy(norm_hbm.at[s, pl.ds(gb, IGR)], nrm_v)

        pltpu.async_copy(xt_hbm.at[c].at[gidx_v.at[0]], rows0, gsem0)

        def pair(k2, _):
            k = k2 * 2
            pltpu.make_async_copy(
                xt_hbm.at[c].at[gidx_v.at[k]], rows0, gsem0).wait()
            pltpu.async_copy(xt_hbm.at[c].at[gidx_v.at[k + 1]], rows1, gsem1)
            scale_scatter(rows0, k)
            pltpu.make_async_copy(
                xt_hbm.at[c].at[gidx_v.at[k + 1]], rows1, gsem1).wait()

            @pl.when(k2 < IGR // 2 - 1)
            def _():
                pltpu.async_copy(
                    xt_hbm.at[c].at[gidx_v.at[k + 2]], rows0, gsem0)

            scale_scatter(rows1, k + 1)
            return 0
        lax.fori_loop(0, IGR // 2, pair, 0)
        return 0
    lax.fori_loop(0, RPS // IGR, ggroup, 0)
    plsc.subcore_barrier()

    def drain_chunk(start, size):
        rb = pl.multiple_of(base + start, 8)
        pltpu.sync_copy(acc.at[pl.ds(rb, size)], drain_v.at[pl.ds(0, size)])

        def rrow(r2, _3):
            for u in range(8):
                drain_v[r2, pl.ds(u * 16, 16)] = jnp.maximum(
                    drain_v[r2, pl.ds(u * 16, 16)], 0.0)
            return 0
        lax.fori_loop(0, size, rrow, 0)
        pltpu.sync_copy(drain_v.at[pl.ds(0, size)], h_hbm.at[c, pl.ds(rb, size), :])

    for t in range(6):
        drain_chunk(t * 80, 80)

    @pl.when(s < NS - 1)
    def _():
        drain_chunk(480, 80)
        drain_chunk(560, NPS - 560)

    @pl.when(s == NS - 1)
    def _():
        drain_chunk(480, N - (NS - 1) * NPS - 480)


_agg_call = pl.kernel(
    _agg_body,
    out_type=jax.ShapeDtypeStruct((2, N, 128), jnp.float32),
    mesh=_mesh,
    scratch_types=[
        pltpu.VMEM_SHARED((NPAD, 128), jnp.float32),
        pltpu.VMEM((IGR, CH), jnp.int32),
        pltpu.VMEM((IGR, CH), jnp.int32),
        pltpu.VMEM((IGR, CH), jnp.float32),
        pltpu.VMEM((CH, 128), jnp.float32),
        pltpu.VMEM((CH, 128), jnp.float32),
        pltpu.VMEM((80, 128), jnp.float32),
        pltpu.SemaphoreType.DMA,
        pltpu.SemaphoreType.DMA,
    ],
)


def kernel(node_features, edge_features, edge_flats, graph,
           W1, root1, b1, W2, root2, b2):
    del graph
    src = edge_flats[0].astype(jnp.int32)
    dst = edge_flats[1].astype(jnp.int32)

    efT3 = edge_features.T.reshape(8, E // 128, 128)
    gidx2d, didx2d = _prep(efT3, src.reshape(E // 128, 128),
                           dst.reshape(E // 128, 128))
    npad = EPS * NS - E
    gidx = jnp.concatenate(
        [gidx2d.reshape(-1), jnp.zeros((npad,), jnp.int32)]).reshape(NS, RPS, CH)
    didx_flat = jnp.concatenate(
        [didx2d.reshape(-1), jnp.full((npad,), N * R, jnp.int32)])
    didx16 = didx_flat.reshape(NS, RPS, CH)
    didx32 = didx_flat.reshape(NS * NC, RPW, CH)
    dst80 = jnp.concatenate(
        [dst, jnp.zeros((npad,), jnp.int32)]).reshape(NS, RPS, CH)

    norm = _cnt_call(didx16, didx32).reshape(NS, RPS, CH)

    w1e = (jnp.concatenate([W1, root1[None]], axis=0)
           .reshape(R + 1, 1, 128, 2, 128).transpose(3, 0, 1, 2, 4))
    xt1 = _mm(node_features.reshape(1, N, 128), w1e, b1.reshape(2, 1, 128))
    h1 = _agg_call(xt1, gidx, dst80, norm)

    w2e = (jnp.concatenate([W2, root2[None]], axis=0)
           .reshape(R + 1, 2, 128, 2, 128).transpose(3, 0, 1, 2, 4))
    xt2 = _mm(h1, w2e, b2.reshape(2, 1, 128))
    h2 = _agg_call(xt2, gidx, dst80, norm)

    return (h2.transpose(1, 0, 2).reshape(N, 256), edge_features)
